# baseline (speedup 1.0000x reference)
import functools

import jax
import jax.numpy as jnp
from jax import lax
from jax.experimental import pallas as pl
from jax.experimental.pallas import tpu as pltpu
from jax.experimental.pallas import tpu_sc as plsc

EMB = 16
CONT = 13
NCAT = 26
NTOK = 1 + CONT + NCAT
B = 16384
NC = 2
NS = 16
NW = NC * NS
ROWS_PER_W = B // NW
R = 32
NCHUNK = ROWS_PER_W // R
IDX_PER_CHUNK = R * NCAT
GB = 104
NGATHER = IDX_PER_CHUNK // GB


def _body(idx_hbm, xc_hbm, wb_hbm, table_hbm, out_hbm,
          idx_v, xc_v, gath_v, out_v, wb_v, sem):
    wid = lax.axis_index("s") * NC + lax.axis_index("c")
    pltpu.sync_copy(wb_hbm, wb_v)

    def chunk(g, carry):
        base = wid * ROWS_PER_W + g * R
        irow = (wid * NCHUNK + g) * NGATHER
        pltpu.sync_copy(idx_hbm.at[pl.ds(irow, NGATHER)], idx_v)
        pltpu.sync_copy(xc_hbm.at[pl.ds(base, R)], xc_v)

        copies = [
            pltpu.async_copy(
                table_hbm.at[idx_v.at[j]],
                gath_v.at[pl.ds(j * GB, GB)],
                sem,
            )
            for j in range(NGATHER)
        ]
        for c in copies:
            c.wait()

        def row(r, carry2):
            out_v[r, 0, :] = wb_v[0, :]
            xr = xc_v[r, :]
            for j in range(CONT):
                s = xr[j]
                out_v[r, 1 + j, :] = wb_v[1 + j, :] * s + wb_v[14 + j, :]
            for c in range(NCAT):
                out_v[r, 14 + c, :] = gath_v[r * NCAT + c, :] + wb_v[27 + c, :]
            return carry2

        lax.fori_loop(0, R, row, 0)
        pltpu.sync_copy(out_v, out_hbm.at[pl.ds(base, R)])
        return carry

    lax.fori_loop(0, NCHUNK, chunk, 0)


@jax.jit
def _tokenize(idx, xc_pad, wb, cat_weights):
    mesh = plsc.VectorSubcoreMesh(core_axis_name="c", subcore_axis_name="s")
    return pl.kernel(
        _body,
        out_type=jax.ShapeDtypeStruct((B, NTOK, EMB), jnp.float32),
        mesh=mesh,
        scratch_types=[
            pltpu.VMEM((NGATHER, GB), jnp.int32),
            pltpu.VMEM((R, EMB), jnp.float32),
            pltpu.VMEM((IDX_PER_CHUNK, EMB), jnp.float32),
            pltpu.VMEM((R, NTOK, EMB), jnp.float32),
            pltpu.VMEM((1 + CONT + CONT + NCAT, EMB), jnp.float32),
            pltpu.SemaphoreType.DMA,
        ],
        compiler_params=pltpu.CompilerParams(use_tc_tiling_on_sc=False),
    )(idx, xc_pad, wb, cat_weights)


def kernel(x, weight, bias, cat_weights):
    offsets = jnp.arange(NCAT, dtype=jnp.int32) * 100000
    idx = (x[:, :NCAT].astype(jnp.int32) + offsets[None]).reshape(-1, GB)
    xc_pad = jnp.concatenate(
        [x[:, NCAT:], jnp.zeros((B, EMB - CONT), jnp.float32)], axis=1)
    wb = jnp.concatenate([weight, bias], axis=0)
    out = _tokenize(idx, xc_pad, wb, cat_weights)
    return out.reshape(B, NTOK * EMB)
